# baseline (speedup 1.0000x reference)
import jax
import jax.numpy as jnp
from jax.experimental import pallas as pl
from jax.experimental.pallas import tpu as pltpu

_N = 20000
_C = 20
_M = 20
_NB = 157
_NBP = 160
_NP = _NB * 128
_CONF_T = 0.5
_IOU_T = 0.5
_NEG = -1e30


def _nms_lazy_kernel(scores_ref, boxes_n4_ref, oy1_ref, ox1_ref, oy2_ref, ox2_ref, os_ref, s3_ref):
    lane_b = jax.lax.broadcasted_iota(jnp.int32, (_C, _NBP), 1)
    lane_i = jax.lax.broadcasted_iota(jnp.int32, (_C, 128), 1)
    slot = jax.lax.broadcasted_iota(jnp.int32, (_C, _M), 1)

    s0 = scores_ref[...]
    s3 = jnp.where(s0 >= _CONF_T, s0, _NEG)
    s3_ref[...] = s3
    b0 = jnp.concatenate(
        [jnp.max(s3, axis=2), jnp.full((_C, _NBP - _NB), _NEG, jnp.float32)],
        axis=1,
    )

    zcm = jnp.zeros((_C, _M), jnp.float32)
    init = (
        jnp.int32(1),
        jnp.zeros((_C, 1), jnp.int32),
        b0,
        zcm, zcm, zcm, zcm,
        zcm,
    )

    def examine(cnt, bmax, ky1, kx1, ky2, kx2, ks):
        m = jnp.max(bmax, axis=1, keepdims=True)
        j = jnp.min(jnp.where(bmax == m, lane_b, _NB - 1), axis=1, keepdims=True)
        keep = m > (_NEG * 0.5)

        blocks = jnp.concatenate(
            [s3_ref[c, pl.ds(j[c, 0], 1), :] for c in range(_C)], axis=0
        )
        pos = jnp.min(jnp.where(blocks == m, lane_i, 127), axis=1, keepdims=True)
        idx = j * 128 + pos

        blocks_new = jnp.where(lane_i == pos, _NEG, blocks)
        for c in range(_C):
            s3_ref[c, pl.ds(j[c, 0], 1), :] = blocks_new[c : c + 1, :]
        bm = jnp.max(blocks_new, axis=1, keepdims=True)
        bmax = jnp.where(lane_b == j, bm, bmax)

        rows = [boxes_n4_ref[pl.ds(idx[c, 0], 1), :] for c in range(_C)]
        sel = jnp.concatenate(rows, axis=0)
        cy1 = sel[:, 0:1]
        cx1 = sel[:, 1:2]
        cy2 = sel[:, 2:3]
        cx2 = sel[:, 3:4]
        ca = jnp.maximum(cy2 - cy1, 0.0) * jnp.maximum(cx2 - cx1, 0.0)

        ka = jnp.maximum(ky2 - ky1, 0.0) * jnp.maximum(kx2 - kx1, 0.0)
        yy1 = jnp.maximum(ky1, cy1)
        xx1 = jnp.maximum(kx1, cx1)
        yy2 = jnp.minimum(ky2, cy2)
        xx2 = jnp.minimum(kx2, cx2)
        inter = jnp.maximum(yy2 - yy1, 0.0) * jnp.maximum(xx2 - xx1, 0.0)
        union = jnp.maximum(ka + ca - inter, 1e-9)
        iou = inter / union
        suppressed = jnp.any(iou > _IOU_T, axis=1, keepdims=True)

        accept = jnp.logical_and(keep, jnp.logical_not(suppressed))
        upd = jnp.logical_and(slot == cnt, accept)
        ky1 = jnp.where(upd, cy1, ky1)
        kx1 = jnp.where(upd, cx1, kx1)
        ky2 = jnp.where(upd, cy2, ky2)
        kx2 = jnp.where(upd, cx2, kx2)
        ks = jnp.where(upd, m, ks)
        cnt = cnt + accept.astype(jnp.int32)
        return cnt, bmax, ky1, kx1, ky2, kx2, ks, keep

    def body_fixed(i, carry):
        return examine(*carry)[:-1]

    carry = jax.lax.fori_loop(0, _M + 3, body_fixed, init[1:], unroll=False)

    m0 = jnp.max(carry[1], axis=1, keepdims=True)
    live0 = jnp.sum(
        jnp.logical_and(carry[0] < _M, m0 > (_NEG * 0.5)).astype(jnp.int32),
        axis=0,
        keepdims=True,
    )

    def cond(carry):
        return carry[0] > 0

    def body(carry):
        cnt, bmax, ky1, kx1, ky2, kx2, ks, keep = examine(*carry[1:])
        live_vec = jnp.logical_and(cnt < _M, keep).astype(jnp.int32)
        live = jnp.sum(live_vec, axis=0, keepdims=True)
        return (live[0, 0], cnt, bmax, ky1, kx1, ky2, kx2, ks)

    out = jax.lax.while_loop(cond, body, (live0[0, 0],) + carry)
    _, _, _, ky1, kx1, ky2, kx2, ks = out
    oy1_ref[...] = ky1
    ox1_ref[...] = kx1
    oy2_ref[...] = ky2
    ox2_ref[...] = kx2
    os_ref[...] = ks


def kernel(boxes, box_scores):
    scores_t = box_scores.T
    scores_p = jnp.pad(scores_t, ((0, 0), (0, _NP - _N))).reshape(_C, _NB, 128)
    oy1, ox1, oy2, ox2, osc = pl.pallas_call(
        _nms_lazy_kernel,
        out_shape=[jax.ShapeDtypeStruct((_C, _M), jnp.float32)] * 5,
        scratch_shapes=[pltpu.VMEM((_C, _NB, 128), jnp.float32)],
    )(scores_p, boxes)
    box_array = jnp.stack([oy1, ox1, oy2, ox2], axis=-1).reshape(-1, 4)
    score_array = osc.reshape(-1)
    class_array = jnp.repeat(jnp.arange(_C, dtype=jnp.int32), _M)
    return box_array, score_array, class_array


# speedup vs baseline: 1.0229x; 1.0229x over previous
import jax
import jax.numpy as jnp
from jax.experimental import pallas as pl
from jax.experimental.pallas import tpu as pltpu

_N = 20000
_C = 20
_M = 20
_NB = 157
_NBP = 160
_NP = _NB * 128
_CONF_T = 0.5
_IOU_T = 0.5
_NEG = -1e30


def _nms_lazy_kernel(scores_ref, boxes_n4_ref, oy1_ref, ox1_ref, oy2_ref, ox2_ref, os_ref, s3_ref):
    lane_b = jax.lax.broadcasted_iota(jnp.int32, (_C, _NBP), 1)
    lane_i = jax.lax.broadcasted_iota(jnp.int32, (_C, 128), 1)
    slot = jax.lax.broadcasted_iota(jnp.int32, (_C, _M), 1)

    s0 = scores_ref[...]
    s3 = jnp.where(s0 >= _CONF_T, s0, _NEG)
    s3_ref[...] = s3
    b0 = jnp.concatenate(
        [jnp.max(s3, axis=2), jnp.full((_C, _NBP - _NB), _NEG, jnp.float32)],
        axis=1,
    )

    zcm = jnp.zeros((_C, _M), jnp.float32)
    init = (
        jnp.int32(1),
        jnp.zeros((_C, 1), jnp.int32),
        b0,
        zcm, zcm, zcm, zcm,
        zcm,
    )

    def cond(carry):
        return carry[0] > 0

    def body(carry):
        _, cnt, bmax, ky1, kx1, ky2, kx2, ks = carry
        m = jnp.max(bmax, axis=1, keepdims=True)
        j = jnp.min(jnp.where(bmax == m, lane_b, _NB - 1), axis=1, keepdims=True)
        keep = m > (_NEG * 0.5)

        blocks = jnp.concatenate(
            [s3_ref[c, pl.ds(j[c, 0], 1), :] for c in range(_C)], axis=0
        )
        pos = jnp.min(jnp.where(blocks == m, lane_i, 127), axis=1, keepdims=True)
        idx = jnp.minimum(j * 128 + pos, _N - 1)

        blocks_new = jnp.where(lane_i == pos, _NEG, blocks)
        for c in range(_C):
            s3_ref[c, pl.ds(j[c, 0], 1), :] = blocks_new[c : c + 1, :]
        bm = jnp.max(blocks_new, axis=1, keepdims=True)
        bmax = jnp.where(lane_b == j, bm, bmax)

        rows = [boxes_n4_ref[pl.ds(idx[c, 0], 1), :] for c in range(_C)]
        sel = jnp.concatenate(rows, axis=0)
        cy1 = sel[:, 0:1]
        cx1 = sel[:, 1:2]
        cy2 = sel[:, 2:3]
        cx2 = sel[:, 3:4]
        ca = jnp.maximum(cy2 - cy1, 0.0) * jnp.maximum(cx2 - cx1, 0.0)

        ka = jnp.maximum(ky2 - ky1, 0.0) * jnp.maximum(kx2 - kx1, 0.0)
        yy1 = jnp.maximum(ky1, cy1)
        xx1 = jnp.maximum(kx1, cx1)
        yy2 = jnp.minimum(ky2, cy2)
        xx2 = jnp.minimum(kx2, cx2)
        inter = jnp.maximum(yy2 - yy1, 0.0) * jnp.maximum(xx2 - xx1, 0.0)
        union = jnp.maximum(ka + ca - inter, 1e-9)
        iou = inter / union
        suppressed = jnp.any(iou > _IOU_T, axis=1, keepdims=True)

        accept = jnp.logical_and(keep, jnp.logical_not(suppressed))
        upd = jnp.logical_and(slot == cnt, accept)
        ky1 = jnp.where(upd, cy1, ky1)
        kx1 = jnp.where(upd, cx1, kx1)
        ky2 = jnp.where(upd, cy2, ky2)
        kx2 = jnp.where(upd, cx2, kx2)
        ks = jnp.where(upd, m, ks)
        cnt = cnt + accept.astype(jnp.int32)

        live_vec = jnp.logical_and(cnt < _M, keep).astype(jnp.int32)
        live = jnp.sum(live_vec, axis=0, keepdims=True)
        return (live[0, 0], cnt, bmax, ky1, kx1, ky2, kx2, ks)

    out = jax.lax.while_loop(cond, body, init)
    _, _, _, ky1, kx1, ky2, kx2, ks = out
    oy1_ref[...] = ky1
    ox1_ref[...] = kx1
    oy2_ref[...] = ky2
    ox2_ref[...] = kx2
    os_ref[...] = ks


def kernel(boxes, box_scores):
    scores_t = box_scores.T
    scores_p = jnp.pad(scores_t, ((0, 0), (0, _NP - _N))).reshape(_C, _NB, 128)
    oy1, ox1, oy2, ox2, osc = pl.pallas_call(
        _nms_lazy_kernel,
        out_shape=[jax.ShapeDtypeStruct((_C, _M), jnp.float32)] * 5,
        scratch_shapes=[pltpu.VMEM((_C, _NB, 128), jnp.float32)],
    )(scores_p, boxes)
    box_array = jnp.stack([oy1, ox1, oy2, ox2], axis=-1).reshape(-1, 4)
    score_array = osc.reshape(-1)
    class_array = jnp.repeat(jnp.arange(_C, dtype=jnp.int32), _M)
    return box_array, score_array, class_array


# speedup vs baseline: 1.0324x; 1.0092x over previous
import jax
import jax.numpy as jnp
from jax.experimental import pallas as pl
from jax.experimental.pallas import tpu as pltpu

_N = 20000
_C = 20
_M = 20
_NB = 157
_NBP = 160
_NP = _NB * 128
_CONF_T = 0.5
_IOU_T = 0.5
_NEG = -1e30
_EXP_BITS = 0x3F000000


def _nms_lazy_kernel(scores_ref, boxes_n4_ref, oy1_ref, ox1_ref, oy2_ref, ox2_ref, os_ref, k3_ref):
    lane_b = jax.lax.broadcasted_iota(jnp.int32, (_C, _NBP), 1)
    lane_i = jax.lax.broadcasted_iota(jnp.int32, (_C, 128), 1)
    pos_i3 = jax.lax.broadcasted_iota(jnp.int32, (_C, _NB, 128), 2)
    slot = jax.lax.broadcasted_iota(jnp.int32, (_C, _M), 1)

    s0 = scores_ref[...]
    mant = jax.lax.bitcast_convert_type(s0, jnp.int32) & 0x7FFFFF
    k3 = jnp.where(s0 >= _CONF_T, (mant << 8) | (127 - pos_i3), -1)
    k3_ref[...] = k3
    kb0 = jnp.max(k3, axis=2)
    blk = jax.lax.broadcasted_iota(jnp.int32, (_C, _NB), 1)
    kb0 = jnp.where(kb0 < 0, -1, (kb0 & -256) | (255 - blk))
    b0 = jnp.concatenate(
        [kb0, jnp.full((_C, _NBP - _NB), -1, jnp.int32)], axis=1
    )

    zcm = jnp.zeros((_C, _M), jnp.float32)
    init = (
        jnp.int32(1),
        jnp.zeros((_C, 1), jnp.int32),
        b0,
        zcm, zcm, zcm, zcm,
        zcm,
    )

    def cond(carry):
        return carry[0] > 0

    def body(carry):
        _, cnt, bmax, ky1, kx1, ky2, kx2, ks = carry
        kbest = jnp.max(bmax, axis=1, keepdims=True)
        keep = kbest >= 0
        j = 255 - (kbest & 255)
        m = jax.lax.bitcast_convert_type(
            ((kbest >> 8) & 0x7FFFFF) | _EXP_BITS, jnp.float32
        )

        blocks = jnp.concatenate(
            [k3_ref[c, pl.ds(j[c, 0], 1), :] for c in range(_C)], axis=0
        )
        kin = jnp.max(blocks, axis=1, keepdims=True)
        pos = 127 - (kin & 255)
        idx = jnp.maximum(j * 128 + pos, 0)

        blocks_new = jnp.where(lane_i == pos, -1, blocks)
        for c in range(_C):
            k3_ref[c, pl.ds(j[c, 0], 1), :] = blocks_new[c : c + 1, :]
        bm = jnp.max(blocks_new, axis=1, keepdims=True)
        bm = jnp.where(bm < 0, -1, (bm & -256) | (255 - j))
        bmax = jnp.where(lane_b == j, bm, bmax)

        rows = [boxes_n4_ref[pl.ds(idx[c, 0], 1), :] for c in range(_C)]
        sel = jnp.concatenate(rows, axis=0)
        cy1 = sel[:, 0:1]
        cx1 = sel[:, 1:2]
        cy2 = sel[:, 2:3]
        cx2 = sel[:, 3:4]
        ca = jnp.maximum(cy2 - cy1, 0.0) * jnp.maximum(cx2 - cx1, 0.0)

        ka = jnp.maximum(ky2 - ky1, 0.0) * jnp.maximum(kx2 - kx1, 0.0)
        yy1 = jnp.maximum(ky1, cy1)
        xx1 = jnp.maximum(kx1, cx1)
        yy2 = jnp.minimum(ky2, cy2)
        xx2 = jnp.minimum(kx2, cx2)
        inter = jnp.maximum(yy2 - yy1, 0.0) * jnp.maximum(xx2 - xx1, 0.0)
        union = jnp.maximum(ka + ca - inter, 1e-9)
        iou = inter / union
        suppressed = jnp.any(iou > _IOU_T, axis=1, keepdims=True)

        accept = jnp.logical_and(keep, jnp.logical_not(suppressed))
        upd = jnp.logical_and(slot == cnt, accept)
        ky1 = jnp.where(upd, cy1, ky1)
        kx1 = jnp.where(upd, cx1, kx1)
        ky2 = jnp.where(upd, cy2, ky2)
        kx2 = jnp.where(upd, cx2, kx2)
        ks = jnp.where(upd, m, ks)
        cnt = cnt + accept.astype(jnp.int32)

        live_vec = jnp.logical_and(cnt < _M, keep).astype(jnp.int32)
        live = jnp.sum(live_vec, axis=0, keepdims=True)
        return (live[0, 0], cnt, bmax, ky1, kx1, ky2, kx2, ks)

    out = jax.lax.while_loop(cond, body, init)
    _, _, _, ky1, kx1, ky2, kx2, ks = out
    oy1_ref[...] = ky1
    ox1_ref[...] = kx1
    oy2_ref[...] = ky2
    ox2_ref[...] = kx2
    os_ref[...] = ks


def kernel(boxes, box_scores):
    scores_t = box_scores.T
    scores_p = jnp.pad(scores_t, ((0, 0), (0, _NP - _N))).reshape(_C, _NB, 128)
    oy1, ox1, oy2, ox2, osc = pl.pallas_call(
        _nms_lazy_kernel,
        out_shape=[jax.ShapeDtypeStruct((_C, _M), jnp.float32)] * 5,
        scratch_shapes=[pltpu.VMEM((_C, _NB, 128), jnp.int32)],
    )(scores_p, boxes)
    box_array = jnp.stack([oy1, ox1, oy2, ox2], axis=-1).reshape(-1, 4)
    score_array = osc.reshape(-1)
    class_array = jnp.repeat(jnp.arange(_C, dtype=jnp.int32), _M)
    return box_array, score_array, class_array
